# baseline (speedup 1.0000x reference)
import functools

import jax
import jax.numpy as jnp
from jax import lax
from jax.experimental import pallas as pl
from jax.experimental.pallas import tpu as pltpu
from jax.experimental.pallas import tpu_sc as plsc

N = 10000
E = 320000
IN = 128
OUT = 128
NB = 2
T = 2
R = 4
RTT = R * T * T

NC = 2
NS = 16
NW = NC * NS
CH = 128
NCK0 = 79
NCK1 = 79
MAXC = max(NCK0, NCK1)
E_PAD = NS * (NCK0 + NCK1) * CH
RPT = 632
NP = RPT * NS
NPACK = NP // 4
BN = 400
HL = RTT * OUT


def _wbig_body(w_ref, wct0_ref, wct1_ref, wbig_ref):
    oo = lax.broadcasted_iota(jnp.int32, (IN, HL), 0)
    cc = lax.broadcasted_iota(jnp.int32, (IN, HL), 1)
    e = jnp.where(oo == (cc >> 4), 1.0, 0.0).astype(jnp.float32)
    w0r = lax.dot_general(w_ref[:, :IN], e, (((0,), (0,)), ((), ())),
                          preferred_element_type=jnp.float32)
    w1r = lax.dot_general(w_ref[:, IN:], e, (((0,), (0,)), ((), ())),
                          preferred_element_type=jnp.float32)
    wbig_ref[...] = w0r * wct0_ref[...] + w1r * wct1_ref[...]


def _h_body(x_ref, wbig_ref, h_ref):
    h2d = lax.dot_general(x_ref[...], wbig_ref[...],
                          (((1,), (0,)), ((), ())),
                          preferred_element_type=jnp.float32)
    for comp in range(RTT):
        h_ref[:, comp, :] = h2d[:, comp * OUT:(comp + 1) * OUT]


def _final_body(p_ref, b_ref, o_ref):
    o_ref[...] = p_ref[0] + p_ref[1] + b_ref[...]


def _sc_body(edges_hbm, ntp_hbm, h_hbm, out_hbm,
             ebuf, ntp_v, gidx_v, rows_v, agg_sh, fsem, gsem):
    c = lax.axis_index("c")
    s = lax.axis_index("s")
    nck = jnp.where(c == 0, NCK0, NCK1)

    pltpu.sync_copy(ntp_hbm, ntp_v)

    @pl.loop(0, 2 * CH * (OUT // 16))
    def _zero_rows(i):
        r = i // (OUT // 16)
        k = (i % (OUT // 16)) * 16
        rows_v[r, pl.ds(k, 16)] = jnp.zeros((16,), jnp.float32)

    base = s * RPT

    @pl.loop(0, 4)
    def _zero_agg(j):
        pltpu.sync_copy(rows_v.at[pl.ds(0, CH)],
                        agg_sh.at[pl.ds(base + j * CH, CH)])

    pltpu.sync_copy(rows_v.at[pl.ds(0, RPT - 4 * CH)],
                    agg_sh.at[pl.ds(base + 4 * CH, RPT - 4 * CH)])
    plsc.subcore_barrier()

    def compute_gidx(b):
        @pl.loop(0, CH // 16)
        def _indices(k):
            sl = pl.ds(k * 16, 16)
            sv = ebuf[b * 3 + 0, sl]
            dv = ebuf[b * 3 + 1, sl]
            ev = ebuf[b * 3 + 2, sl]
            ws = plsc.load_gather(ntp_v, [sv >> 2])
            wd = plsc.load_gather(ntp_v, [dv >> 2])
            nts = (ws >> ((sv & 3) * 8)) & 3
            ntd = (wd >> ((dv & 3) * 8)) & 3
            gidx_v[b, sl] = sv * RTT + nts * (T * R) + ntd * R + ev

    def start_fetch(j, b):
        pltpu.async_copy(edges_hbm.at[c, s, j], ebuf.at[pl.ds(b * 3, 3)], fsem)

    def wait_fetch(j, b):
        pltpu.make_async_copy(edges_hbm.at[c, s, j], ebuf.at[pl.ds(b * 3, 3)],
                              fsem).wait()

    def start_gather(b):
        pltpu.async_copy(h_hbm.at[gidx_v.at[b]],
                         rows_v.at[pl.ds(b * CH, CH)], gsem)

    def wait_gather(b):
        pltpu.make_async_copy(h_hbm.at[gidx_v.at[b]],
                              rows_v.at[pl.ds(b * CH, CH)], gsem).wait()

    def scatter_add(b):
        pltpu.sync_copy(rows_v.at[pl.ds(b * CH, CH)],
                        agg_sh.at[ebuf.at[b * 3 + 1]], add=True)

    pltpu.sync_copy(edges_hbm.at[c, s, 0], ebuf.at[pl.ds(0, 3)])
    compute_gidx(0)
    start_gather(0)
    start_fetch(1, 1)

    @pl.loop(0, nck - 1)
    def _chunks(j):
        b = lax.rem(j, 2)
        bn = 1 - b
        jn = j + 1
        wait_fetch(jn, bn)
        compute_gidx(bn)
        wait_gather(b)
        start_gather(bn)
        scatter_add(b)

        @pl.when(jn < nck - 1)
        def _():
            start_fetch(j + 2, b)

    b_last = lax.rem(nck - 1, 2)
    wait_gather(b_last)
    scatter_add(b_last)

    plsc.subcore_barrier()
    pltpu.sync_copy(agg_sh.at[pl.ds(base, RPT)],
                    out_hbm.at[c, pl.ds(base, RPT)])


def _make_sc_kernel():
    return functools.partial(
        pl.kernel,
        out_type=jax.ShapeDtypeStruct((NC, NP, OUT), jnp.float32),
        mesh=plsc.VectorSubcoreMesh(core_axis_name="c", subcore_axis_name="s",
                                    num_cores=NC, num_subcores=NS),
        scratch_types=[
            pltpu.VMEM((6, CH), jnp.int32),
            pltpu.VMEM((NPACK,), jnp.int32),
            pltpu.VMEM((2, CH), jnp.int32),
            pltpu.VMEM((2 * CH, OUT), jnp.float32),
            pltpu.VMEM_SHARED((NP, OUT), jnp.float32),
            pltpu.SemaphoreType.DMA,
            pltpu.SemaphoreType.DMA,
        ],
        compiler_params=pltpu.CompilerParams(needs_layout_passes=False),
    )(_sc_body)


def kernel(x, node_type, edge_index, edge_type, weight, w_comp, bias):
    src = edge_index[0]
    dst = edge_index[1]
    pad = E_PAD - E

    def layout(a):
        c0 = a[:NS * NCK0 * CH].reshape(NS, NCK0, CH)
        c1 = a[NS * NCK0 * CH:].reshape(NS, NCK1, CH)
        c0 = jnp.pad(c0, ((0, 0), (0, MAXC - NCK0), (0, 0)))
        c1 = jnp.pad(c1, ((0, 0), (0, MAXC - NCK1), (0, 0)))
        return jnp.stack([c0, c1], axis=0)

    src_p = layout(jnp.concatenate([src, jnp.zeros((pad,), jnp.int32)]))
    dst_p = layout(jnp.concatenate([dst, jnp.full((pad,), NP - 1, jnp.int32)]))
    et_p = layout(jnp.concatenate([edge_type, jnp.zeros((pad,), jnp.int32)]))
    edges_p = jnp.stack([src_p, dst_p, et_p], axis=3)

    nt_pad = jnp.concatenate([node_type, jnp.zeros((NP - N,), jnp.int32)])
    ntp = lax.bitcast_convert_type(
        nt_pad.astype(jnp.int8).reshape(NPACK, 4), jnp.int32)

    w2d = weight.reshape(OUT, NB * IN)
    wct0 = jnp.tile(w_comp[:, 0], OUT).reshape(1, HL)
    wct1 = jnp.tile(w_comp[:, 1], OUT).reshape(1, HL)

    wbig = pl.pallas_call(
        _wbig_body,
        in_specs=[
            pl.BlockSpec((OUT, NB * IN), lambda: (0, 0)),
            pl.BlockSpec((1, HL), lambda: (0, 0)),
            pl.BlockSpec((1, HL), lambda: (0, 0)),
        ],
        out_specs=pl.BlockSpec((IN, HL), lambda: (0, 0)),
        out_shape=jax.ShapeDtypeStruct((IN, HL), jnp.float32),
    )(w2d, wct0, wct1)

    h = pl.pallas_call(
        _h_body,
        grid=(N // BN,),
        in_specs=[
            pl.BlockSpec((BN, IN), lambda i: (i, 0)),
            pl.BlockSpec((IN, HL), lambda i: (0, 0)),
        ],
        out_specs=pl.BlockSpec((BN, RTT, OUT), lambda i: (i, 0, 0)),
        out_shape=jax.ShapeDtypeStruct((N, RTT, OUT), jnp.float32),
    )(x, wbig)
    h_flat = h.reshape(RTT * N, OUT)

    partial = _make_sc_kernel()(edges_p, ntp, h_flat)

    out = pl.pallas_call(
        _final_body,
        grid=(N // BN,),
        in_specs=[
            pl.BlockSpec((NC, BN, OUT), lambda i: (0, i, 0)),
            pl.BlockSpec((1, OUT), lambda i: (0, 0)),
        ],
        out_specs=pl.BlockSpec((BN, OUT), lambda i: (i, 0)),
        out_shape=jax.ShapeDtypeStruct((N, OUT), jnp.float32),
    )(partial, bias)
    return out


# speedup vs baseline: 1.0519x; 1.0519x over previous
import functools

import jax
import jax.numpy as jnp
from jax import lax
from jax.experimental import pallas as pl
from jax.experimental.pallas import tpu as pltpu
from jax.experimental.pallas import tpu_sc as plsc

N = 10000
E = 320000
IN = 128
OUT = 128
NB = 2
T = 2
R = 4
RTT = R * T * T

NC = 2
NS = 16
NW = NC * NS
CH = 128
NCHUNK = 79
E_PAD = NW * NCHUNK * CH
RPT = 632
NP = RPT * NS
NPACK = NP // 4
BN = 400
HL = RTT * OUT


def _wbig_body(w_ref, wct0_ref, wct1_ref, wbig_ref):
    oo = lax.broadcasted_iota(jnp.int32, (IN, HL), 0)
    cc = lax.broadcasted_iota(jnp.int32, (IN, HL), 1)
    e = jnp.where(oo == (cc >> 4), 1.0, 0.0).astype(jnp.float32)
    w0r = lax.dot_general(w_ref[:, :IN], e, (((0,), (0,)), ((), ())),
                          preferred_element_type=jnp.float32)
    w1r = lax.dot_general(w_ref[:, IN:], e, (((0,), (0,)), ((), ())),
                          preferred_element_type=jnp.float32)
    wbig_ref[...] = w0r * wct0_ref[...] + w1r * wct1_ref[...]


def _h_body(x_ref, wbig_ref, h_ref):
    h2d = lax.dot_general(x_ref[...], wbig_ref[...],
                          (((1,), (0,)), ((), ())),
                          preferred_element_type=jnp.float32)
    for comp in range(RTT):
        h_ref[:, comp, :] = h2d[:, comp * OUT:(comp + 1) * OUT]


def _final_body(p_ref, b_ref, o_ref):
    o_ref[...] = p_ref[0] + p_ref[1] + b_ref[...]


def _sc_body(edges_hbm, ntp_hbm, h_hbm, out_hbm,
             ebuf, ntp_v, gidx_v, rows_v, agg_sh, fsem, gsem):
    c = lax.axis_index("c")
    s = lax.axis_index("s")
    wid = s * NC + c

    pltpu.sync_copy(ntp_hbm, ntp_v)

    @pl.loop(0, 2 * CH * (OUT // 16))
    def _zero_rows(i):
        r = i // (OUT // 16)
        k = (i % (OUT // 16)) * 16
        rows_v[r, pl.ds(k, 16)] = jnp.zeros((16,), jnp.float32)

    base = s * RPT

    @pl.loop(0, 4)
    def _zero_agg(j):
        pltpu.sync_copy(rows_v.at[pl.ds(0, CH)],
                        agg_sh.at[pl.ds(base + j * CH, CH)])

    pltpu.sync_copy(rows_v.at[pl.ds(0, RPT - 4 * CH)],
                    agg_sh.at[pl.ds(base + 4 * CH, RPT - 4 * CH)])
    plsc.subcore_barrier()

    def compute_gidx(b):
        @pl.loop(0, CH // 16)
        def _indices(k):
            sl = pl.ds(k * 16, 16)
            sv = ebuf[b * 3 + 0, sl]
            dv = ebuf[b * 3 + 1, sl]
            ev = ebuf[b * 3 + 2, sl]
            ws = plsc.load_gather(ntp_v, [sv >> 2])
            wd = plsc.load_gather(ntp_v, [dv >> 2])
            nts = (ws >> ((sv & 3) * 8)) & 3
            ntd = (wd >> ((dv & 3) * 8)) & 3
            gidx_v[b, sl] = sv * RTT + nts * (T * R) + ntd * R + ev

    def start_fetch(j, b):
        pltpu.async_copy(edges_hbm.at[wid, j], ebuf.at[pl.ds(b * 3, 3)], fsem)

    def wait_fetch(j, b):
        pltpu.make_async_copy(edges_hbm.at[wid, j], ebuf.at[pl.ds(b * 3, 3)],
                              fsem).wait()

    def start_gather(b):
        pltpu.async_copy(h_hbm.at[gidx_v.at[b]],
                         rows_v.at[pl.ds(b * CH, CH)], gsem)

    def wait_gather(b):
        pltpu.make_async_copy(h_hbm.at[gidx_v.at[b]],
                              rows_v.at[pl.ds(b * CH, CH)], gsem).wait()

    def scatter_add(b):
        pltpu.sync_copy(rows_v.at[pl.ds(b * CH, CH)],
                        agg_sh.at[ebuf.at[b * 3 + 1]], add=True)

    pltpu.sync_copy(edges_hbm.at[wid, 0], ebuf.at[pl.ds(0, 3)])
    compute_gidx(0)
    start_gather(0)
    start_fetch(1, 1)

    @pl.loop(0, NCHUNK - 1)
    def _chunks(j):
        b = lax.rem(j, 2)
        bn = 1 - b
        jn = j + 1
        wait_fetch(jn, bn)
        compute_gidx(bn)
        wait_gather(b)
        start_gather(bn)
        scatter_add(b)

        @pl.when(jn < NCHUNK - 1)
        def _():
            start_fetch(j + 2, b)

    b_last = (NCHUNK - 1) % 2
    wait_gather(b_last)
    scatter_add(b_last)

    plsc.subcore_barrier()
    pltpu.sync_copy(agg_sh.at[pl.ds(base, RPT)],
                    out_hbm.at[c, pl.ds(base, RPT)])


def _make_sc_kernel():
    return functools.partial(
        pl.kernel,
        out_type=jax.ShapeDtypeStruct((NC, NP, OUT), jnp.float32),
        mesh=plsc.VectorSubcoreMesh(core_axis_name="c", subcore_axis_name="s",
                                    num_cores=NC, num_subcores=NS),
        scratch_types=[
            pltpu.VMEM((6, CH), jnp.int32),
            pltpu.VMEM((NPACK,), jnp.int32),
            pltpu.VMEM((2, CH), jnp.int32),
            pltpu.VMEM((2 * CH, OUT), jnp.float32),
            pltpu.VMEM_SHARED((NP, OUT), jnp.float32),
            pltpu.SemaphoreType.DMA,
            pltpu.SemaphoreType.DMA,
        ],
        compiler_params=pltpu.CompilerParams(needs_layout_passes=False),
    )(_sc_body)


def kernel(x, node_type, edge_index, edge_type, weight, w_comp, bias):
    src = edge_index[0]
    dst = edge_index[1]
    pad = E_PAD - E

    src_p = jnp.concatenate(
        [src, jnp.zeros((pad,), jnp.int32)]).reshape(NW, NCHUNK, CH)
    dst_p = jnp.concatenate(
        [dst, jnp.full((pad,), NP - 1, jnp.int32)]).reshape(NW, NCHUNK, CH)
    et_p = jnp.concatenate(
        [edge_type, jnp.zeros((pad,), jnp.int32)]).reshape(NW, NCHUNK, CH)
    edges_p = jnp.stack([src_p, dst_p, et_p], axis=2)

    nt_pad = jnp.concatenate([node_type, jnp.zeros((NP - N,), jnp.int32)])
    ntp = lax.bitcast_convert_type(
        nt_pad.astype(jnp.int8).reshape(NPACK, 4), jnp.int32)

    w2d = weight.reshape(OUT, NB * IN)
    wct0 = jnp.tile(w_comp[:, 0], OUT).reshape(1, HL)
    wct1 = jnp.tile(w_comp[:, 1], OUT).reshape(1, HL)

    wbig = pl.pallas_call(
        _wbig_body,
        in_specs=[
            pl.BlockSpec((OUT, NB * IN), lambda: (0, 0)),
            pl.BlockSpec((1, HL), lambda: (0, 0)),
            pl.BlockSpec((1, HL), lambda: (0, 0)),
        ],
        out_specs=pl.BlockSpec((IN, HL), lambda: (0, 0)),
        out_shape=jax.ShapeDtypeStruct((IN, HL), jnp.float32),
    )(w2d, wct0, wct1)

    h = pl.pallas_call(
        _h_body,
        grid=(N // BN,),
        in_specs=[
            pl.BlockSpec((BN, IN), lambda i: (i, 0)),
            pl.BlockSpec((IN, HL), lambda i: (0, 0)),
        ],
        out_specs=pl.BlockSpec((BN, RTT, OUT), lambda i: (i, 0, 0)),
        out_shape=jax.ShapeDtypeStruct((N, RTT, OUT), jnp.float32),
    )(x, wbig)
    h_flat = h.reshape(RTT * N, OUT)

    partial = _make_sc_kernel()(edges_p, ntp, h_flat)

    out = pl.pallas_call(
        _final_body,
        grid=(N // BN,),
        in_specs=[
            pl.BlockSpec((NC, BN, OUT), lambda i: (0, i, 0)),
            pl.BlockSpec((1, OUT), lambda i: (0, 0)),
        ],
        out_specs=pl.BlockSpec((BN, OUT), lambda i: (i, 0)),
        out_shape=jax.ShapeDtypeStruct((N, OUT), jnp.float32),
    )(partial, bias)
    return out


# speedup vs baseline: 1.1294x; 1.0737x over previous
import functools

import jax
import jax.numpy as jnp
from jax import lax
from jax.experimental import pallas as pl
from jax.experimental.pallas import tpu as pltpu
from jax.experimental.pallas import tpu_sc as plsc

N = 10000
E = 320000
IN = 128
OUT = 128
NB = 2
T = 2
R = 4
RTT = R * T * T

NC = 2
NS = 16
NW = NC * NS
CH = 128
NCHUNK = 79
E_PAD = NW * NCHUNK * CH
RPT = 632
NP = RPT * NS
NPACK = NP // 4
BN = 400
BF = 2000
HL = RTT * OUT


def _wbig_body(w_ref, wct0_ref, wct1_ref, wbig_ref):
    oo = lax.broadcasted_iota(jnp.int32, (IN, HL), 0)
    cc = lax.broadcasted_iota(jnp.int32, (IN, HL), 1)
    e = jnp.where(oo == (cc >> 4), 1.0, 0.0).astype(jnp.float32)
    w0r = lax.dot_general(w_ref[:, :IN], e, (((0,), (0,)), ((), ())),
                          preferred_element_type=jnp.float32)
    w1r = lax.dot_general(w_ref[:, IN:], e, (((0,), (0,)), ((), ())),
                          preferred_element_type=jnp.float32)
    wbig_ref[...] = w0r * wct0_ref[...] + w1r * wct1_ref[...]


def _h_body(x_ref, wbig_ref, h_ref):
    h2d = lax.dot_general(x_ref[...].astype(jnp.bfloat16),
                          wbig_ref[...].astype(jnp.bfloat16),
                          (((1,), (0,)), ((), ())),
                          preferred_element_type=jnp.float32)
    h_ref[...] = h2d.reshape(BN, RTT, OUT)


def _final_body(p_ref, b_ref, o_ref):
    o_ref[...] = p_ref[0] + p_ref[1] + b_ref[...]


def _sc_body(edges_hbm, ntp_hbm, h_hbm, out_hbm,
             ebuf, ntp_v, gidx_v, rows_v, agg_sh, fsem, gsem):
    c = lax.axis_index("c")
    s = lax.axis_index("s")
    wid = s * NC + c

    pltpu.sync_copy(ntp_hbm, ntp_v)

    @pl.loop(0, 2 * CH * (OUT // 16))
    def _zero_rows(i):
        r = i // (OUT // 16)
        k = (i % (OUT // 16)) * 16
        rows_v[r, pl.ds(k, 16)] = jnp.zeros((16,), jnp.float32)

    base = s * RPT

    @pl.loop(0, 4)
    def _zero_agg(j):
        pltpu.sync_copy(rows_v.at[pl.ds(0, CH)],
                        agg_sh.at[pl.ds(base + j * CH, CH)])

    pltpu.sync_copy(rows_v.at[pl.ds(0, RPT - 4 * CH)],
                    agg_sh.at[pl.ds(base + 4 * CH, RPT - 4 * CH)])
    plsc.subcore_barrier()

    def compute_gidx(b):
        @pl.loop(0, CH // 16)
        def _indices(k):
            sl = pl.ds(k * 16, 16)
            sv = ebuf[b * 3 + 0, sl]
            dv = ebuf[b * 3 + 1, sl]
            ev = ebuf[b * 3 + 2, sl]
            ws = plsc.load_gather(ntp_v, [sv >> 2])
            wd = plsc.load_gather(ntp_v, [dv >> 2])
            nts = (ws >> ((sv & 3) * 8)) & 3
            ntd = (wd >> ((dv & 3) * 8)) & 3
            gidx_v[b, sl] = sv * RTT + nts * (T * R) + ntd * R + ev

    def start_fetch(j, b):
        pltpu.async_copy(edges_hbm.at[wid, j], ebuf.at[pl.ds(b * 3, 3)], fsem)

    def wait_fetch(j, b):
        pltpu.make_async_copy(edges_hbm.at[wid, j], ebuf.at[pl.ds(b * 3, 3)],
                              fsem).wait()

    def start_gather(b):
        pltpu.async_copy(h_hbm.at[gidx_v.at[b]],
                         rows_v.at[pl.ds(b * CH, CH)], gsem)

    def wait_gather(b):
        pltpu.make_async_copy(h_hbm.at[gidx_v.at[b]],
                              rows_v.at[pl.ds(b * CH, CH)], gsem).wait()

    def scatter_add(b):
        pltpu.sync_copy(rows_v.at[pl.ds(b * CH, CH)],
                        agg_sh.at[ebuf.at[b * 3 + 1]], add=True)

    pltpu.sync_copy(edges_hbm.at[wid, 0], ebuf.at[pl.ds(0, 3)])
    compute_gidx(0)
    start_gather(0)
    start_fetch(1, 1)

    @pl.loop(0, NCHUNK - 1)
    def _chunks(j):
        b = lax.rem(j, 2)
        bn = 1 - b
        jn = j + 1
        wait_fetch(jn, bn)
        compute_gidx(bn)
        wait_gather(b)
        start_gather(bn)
        scatter_add(b)

        @pl.when(jn < NCHUNK - 1)
        def _():
            start_fetch(j + 2, b)

    b_last = (NCHUNK - 1) % 2
    wait_gather(b_last)
    scatter_add(b_last)

    plsc.subcore_barrier()
    pltpu.sync_copy(agg_sh.at[pl.ds(base, RPT)],
                    out_hbm.at[c, pl.ds(base, RPT)])


def _make_sc_kernel():
    return functools.partial(
        pl.kernel,
        out_type=jax.ShapeDtypeStruct((NC, NP, OUT), jnp.float32),
        mesh=plsc.VectorSubcoreMesh(core_axis_name="c", subcore_axis_name="s",
                                    num_cores=NC, num_subcores=NS),
        scratch_types=[
            pltpu.VMEM((6, CH), jnp.int32),
            pltpu.VMEM((NPACK,), jnp.int32),
            pltpu.VMEM((2, CH), jnp.int32),
            pltpu.VMEM((2 * CH, OUT), jnp.float32),
            pltpu.VMEM_SHARED((NP, OUT), jnp.float32),
            pltpu.SemaphoreType.DMA,
            pltpu.SemaphoreType.DMA,
        ],
        compiler_params=pltpu.CompilerParams(needs_layout_passes=False),
    )(_sc_body)


def kernel(x, node_type, edge_index, edge_type, weight, w_comp, bias):
    src = edge_index[0]
    dst = edge_index[1]
    pad = E_PAD - E

    src_p = jnp.concatenate(
        [src, jnp.zeros((pad,), jnp.int32)]).reshape(NW, NCHUNK, CH)
    dst_p = jnp.concatenate(
        [dst, jnp.full((pad,), NP - 1, jnp.int32)]).reshape(NW, NCHUNK, CH)
    et_p = jnp.concatenate(
        [edge_type, jnp.zeros((pad,), jnp.int32)]).reshape(NW, NCHUNK, CH)
    edges_p = jnp.stack([src_p, dst_p, et_p], axis=2)

    nt_pad = jnp.concatenate([node_type, jnp.zeros((NP - N,), jnp.int32)])
    ntp = lax.bitcast_convert_type(
        nt_pad.astype(jnp.int8).reshape(NPACK, 4), jnp.int32)

    w2d = weight.reshape(OUT, NB * IN)
    wct0 = jnp.tile(w_comp[:, 0], OUT).reshape(1, HL)
    wct1 = jnp.tile(w_comp[:, 1], OUT).reshape(1, HL)

    wbig = pl.pallas_call(
        _wbig_body,
        in_specs=[
            pl.BlockSpec((OUT, NB * IN), lambda: (0, 0)),
            pl.BlockSpec((1, HL), lambda: (0, 0)),
            pl.BlockSpec((1, HL), lambda: (0, 0)),
        ],
        out_specs=pl.BlockSpec((IN, HL), lambda: (0, 0)),
        out_shape=jax.ShapeDtypeStruct((IN, HL), jnp.float32),
    )(w2d, wct0, wct1)

    h = pl.pallas_call(
        _h_body,
        grid=(N // BN,),
        in_specs=[
            pl.BlockSpec((BN, IN), lambda i: (i, 0)),
            pl.BlockSpec((IN, HL), lambda i: (0, 0)),
        ],
        out_specs=pl.BlockSpec((BN, RTT, OUT), lambda i: (i, 0, 0)),
        out_shape=jax.ShapeDtypeStruct((N, RTT, OUT), jnp.float32),
    )(x, wbig)
    h_flat = h.reshape(RTT * N, OUT)

    partial = _make_sc_kernel()(edges_p, ntp, h_flat)

    out = pl.pallas_call(
        _final_body,
        grid=(N // BF,),
        in_specs=[
            pl.BlockSpec((NC, BF, OUT), lambda i: (0, i, 0)),
            pl.BlockSpec((1, OUT), lambda i: (0, 0)),
        ],
        out_specs=pl.BlockSpec((BF, OUT), lambda i: (i, 0)),
        out_shape=jax.ShapeDtypeStruct((N, OUT), jnp.float32),
    )(partial, bias)
    return out
